# Initial kernel scaffold; baseline (speedup 1.0000x reference)
import jax
import jax.numpy as jnp
from jax.experimental import pallas as pl


def kernel(input_ids, weight):
    raise NotImplementedError("write your pallas kernel here")


# speedup vs baseline: 3.2272x; 3.2272x over previous
import functools

import jax
import jax.numpy as jnp
from jax import lax
from jax.experimental import pallas as pl
from jax.experimental.pallas import tpu as pltpu
from jax.experimental.pallas import tpu_sc as plsc

NC = 2
NS = 16
NW = NC * NS
GW = 128
CH = 8
GROUP = CH * GW


def _sc_gather(idx2d, weight):
  R = idx2d.shape[0]
  D = weight.shape[1]
  rows_per_w = R // NW
  n_groups = rows_per_w // CH

  mesh = plsc.VectorSubcoreMesh(core_axis_name="c", subcore_axis_name="s")

  @functools.partial(
      pl.kernel,
      mesh=mesh,
      compiler_params=pltpu.CompilerParams(use_tc_tiling_on_sc=False),
      out_type=jax.ShapeDtypeStruct((R, GW, D), jnp.float32),
      scratch_types=[
          pltpu.VMEM((CH, GW), jnp.int32),
          pltpu.VMEM((CH, GW, D), jnp.float32),
          pltpu.SemaphoreType.DMA,
      ],
  )
  def k(table_hbm, idx_hbm, out_hbm, idx_v, rows_v, sem):
    wid = lax.axis_index("s") * NC + lax.axis_index("c")
    base = wid * rows_per_w

    def body(g, carry):
      row0 = base + g * CH
      pltpu.sync_copy(idx_hbm.at[pl.ds(row0, CH)], idx_v)
      copies = []
      for j in range(CH):
        copies.append(
            pltpu.async_copy(table_hbm.at[idx_v.at[j]], rows_v.at[j], sem))
      for c in copies:
        c.wait()
      pltpu.sync_copy(rows_v, out_hbm.at[pl.ds(row0, CH)])
      return carry

    lax.fori_loop(0, n_groups, body, 0)

  return k(weight, idx2d)


def kernel(input_ids, weight):
  S, T = input_ids.shape
  D = weight.shape[1]
  idx2d = input_ids.reshape(-1, GW).astype(jnp.int32)
  out = _sc_gather(idx2d, weight)
  return out.reshape(S, T, D)


# speedup vs baseline: 4.2153x; 1.3062x over previous
import functools

import jax
import jax.numpy as jnp
from jax import lax
from jax.experimental import pallas as pl
from jax.experimental.pallas import tpu as pltpu
from jax.experimental.pallas import tpu_sc as plsc

NC = 2
NS = 16
NW = NC * NS
CH = 8


def _sc_gather(idx, weight):
  S, T = idx.shape
  D = weight.shape[1]
  rows_per_w = S // NW
  n_groups = rows_per_w // CH

  mesh = plsc.VectorSubcoreMesh(core_axis_name="c", subcore_axis_name="s")

  @functools.partial(
      pl.kernel,
      mesh=mesh,
      compiler_params=pltpu.CompilerParams(use_tc_tiling_on_sc=False),
      out_type=jax.ShapeDtypeStruct((S, T, D), jnp.float32),
      scratch_types=[
          pltpu.VMEM((CH, T), jnp.int32),
          pltpu.VMEM((CH, T, D), jnp.float32),
          pltpu.SemaphoreType.DMA,
      ],
  )
  def k(table_hbm, idx_hbm, out_hbm, idx_v, rows_v, sem):
    wid = lax.axis_index("s") * NC + lax.axis_index("c")
    base = wid * rows_per_w

    def body(g, carry):
      row0 = base + g * CH
      pltpu.sync_copy(idx_hbm.at[pl.ds(row0, CH)], idx_v)
      copies = []
      for j in range(CH):
        copies.append(
            pltpu.async_copy(table_hbm.at[idx_v.at[j]], rows_v.at[j], sem))
      for c in copies:
        c.wait()
      pltpu.sync_copy(rows_v, out_hbm.at[pl.ds(row0, CH)])
      return carry

    lax.fori_loop(0, n_groups, body, 0)

  return k(weight, idx)


def kernel(input_ids, weight):
  return _sc_gather(input_ids.astype(jnp.int32), weight)


# speedup vs baseline: 4.4332x; 1.0517x over previous
import functools

import jax
import jax.numpy as jnp
from jax import lax
from jax.experimental import pallas as pl
from jax.experimental.pallas import tpu as pltpu
from jax.experimental.pallas import tpu_sc as plsc

NC = 2
NS = 16
NW = NC * NS
CH = 8
NB = 2


def _sc_gather(idx, weight):
  S, T = idx.shape
  D = weight.shape[1]
  rows_per_w = S // NW
  n_groups = rows_per_w // CH

  mesh = plsc.VectorSubcoreMesh(core_axis_name="c", subcore_axis_name="s")

  @functools.partial(
      pl.kernel,
      mesh=mesh,
      compiler_params=pltpu.CompilerParams(use_tc_tiling_on_sc=False),
      out_type=jax.ShapeDtypeStruct((S, T, D), jnp.float32),
      scratch_types=[
          pltpu.VMEM((CH, T), jnp.int32),
          pltpu.VMEM((CH, T), jnp.int32),
          pltpu.VMEM((CH, T, D), jnp.float32),
          pltpu.VMEM((CH, T, D), jnp.float32),
          pltpu.SemaphoreType.DMA,
          pltpu.SemaphoreType.DMA,
          pltpu.SemaphoreType.DMA,
          pltpu.SemaphoreType.DMA,
      ],
  )
  def k(table_hbm, idx_hbm, out_hbm, idx0, idx1, rows0, rows1,
        g0s, g1s, o0s, o1s):
    idx_v = [idx0, idx1]
    rows_v = [rows0, rows1]
    gsems = [g0s, g1s]
    osems = [o0s, o1s]
    wid = lax.axis_index("s") * NC + lax.axis_index("c")
    base = wid * rows_per_w

    def idx_slice(g):
      return idx_hbm.at[pl.ds(base + g * CH, CH)]

    def out_slice(g):
      return out_hbm.at[pl.ds(base + g * CH, CH)]

    def fire_gathers(b):
      for j in range(CH):
        pltpu.async_copy(
            table_hbm.at[idx_v[b].at[j]], rows_v[b].at[j], gsems[b])

    def drain_gathers(b):
      for j in range(CH):
        pltpu.make_async_copy(
            table_hbm.at[idx_v[b].at[j]], rows_v[b].at[j], gsems[b]).wait()

    for b in range(NB):
      pltpu.sync_copy(idx_slice(b), idx_v[b])
      fire_gathers(b)

    @pl.loop(0, n_groups - NB, step=NB)
    def body(g0):
      for b in range(NB):
        g = g0 + b
        drain_gathers(b)
        pltpu.async_copy(rows_v[b], out_slice(g), osems[b])
        pltpu.sync_copy(idx_slice(g + NB), idx_v[b])
        pltpu.make_async_copy(rows_v[b], out_slice(g), osems[b]).wait()
        fire_gathers(b)

    for b in range(NB):
      g = n_groups - NB + b
      drain_gathers(b)
      pltpu.sync_copy(rows_v[b], out_slice(g))

  return k(weight, idx)


def kernel(input_ids, weight):
  return _sc_gather(input_ids.astype(jnp.int32), weight)
